# baseline (speedup 1.0000x reference)
import jax
import jax.numpy as jnp
from jax.experimental import pallas as pl
from jax.experimental.pallas import tpu as pltpu

_EMB = 128
_NLAYERS = 3
_G = 16


def _split(x):
    hi = x.astype(jnp.bfloat16)
    lo = (x - hi.astype(jnp.float32)).astype(jnp.bfloat16)
    return hi, lo


def _dot3(ah, al, bh, bl):
    f = jnp.float32
    return (jnp.dot(ah, bh, preferred_element_type=f)
            + jnp.dot(al, bh, preferred_element_type=f)
            + jnp.dot(ah, bl, preferred_element_type=f))


def _gnn_body(xc_ref, pw_ref, pb_ref,
              ws_ref, bs_ref, gamma_ref, beta_ref, alpha_ref, out_ref):
    n = xc_ref.shape[1]
    pw = pw_ref[...]
    pb = pb_ref[...]
    gamma = gamma_ref[...]
    beta = beta_ref[...]
    alpha = alpha_ref[...]

    ir = jax.lax.broadcasted_iota(jnp.int32, (n, n), 0)
    ic = jax.lax.broadcasted_iota(jnp.int32, (n, n), 1)
    eye = (ir == ic).astype(jnp.float32)

    adjs, hs = [], []
    for g in range(_G):
        xg = xc_ref[g]
        xt = jnp.transpose(xg)
        x0c = xg[:, 0:1]
        x1c = xg[:, 1:2]
        x0r = xt[0:1, :]
        x1r = xt[1:2, :]
        d0 = x0c - x0r
        d1 = x1c - x1r
        dist = jnp.sqrt(d0 * d0 + d1 * d1)
        wmat = dist + eye
        dinv_c = jax.lax.rsqrt(jnp.sum(wmat, axis=1, keepdims=True))
        dinv_r = jax.lax.rsqrt(jnp.sum(wmat, axis=0, keepdims=True))
        adjs.append(_split(dinv_c * wmat * dinv_r))
        hs.append(x0c * pw[0:1, :] + x1c * pw[1:2, :] + pb)

    inv_n = 1.0 / n
    for l in range(_NLAYERS):
        wh, wl = _split(ws_ref[l])
        b_l = bs_ref[l:l + 1, :]
        for g in range(_G):
            h = hs[g]
            hh, hl = _split(h)
            xw = _dot3(hh, hl, wh, wl)
            xh, xl = _split(xw)
            ah, al = adjs[g]
            msg = _dot3(ah, al, xh, xl)
            h = jnp.tanh(msg + b_l) + h
            mean = jnp.sum(h, axis=0, keepdims=True) * inv_n
            cent = h - alpha * mean
            var = jnp.sum(cent * cent, axis=0, keepdims=True) * inv_n
            hs[g] = gamma * (cent * jax.lax.rsqrt(var + 1e-5)) + beta

    for g in range(_G):
        out_ref[g] = hs[g]


def kernel(instance, proj_W, proj_b, Ws, bs, gn_gamma, gn_beta, gn_alpha):
    b, n, _ = instance.shape
    full = lambda shape: pl.BlockSpec(shape, lambda g: (0,) * len(shape))
    out = pl.pallas_call(
        _gnn_body,
        grid=(b // _G,),
        in_specs=[
            pl.BlockSpec((_G, n, 2), lambda g: (g, 0, 0)),
            full((2, _EMB)),
            full((1, _EMB)),
            full((_NLAYERS, _EMB, _EMB)),
            full((_NLAYERS, _EMB)),
            full((1, _EMB)),
            full((1, _EMB)),
            full((1, _EMB)),
        ],
        out_specs=pl.BlockSpec((_G, n, _EMB), lambda g: (g, 0, 0)),
        out_shape=jax.ShapeDtypeStruct((b, n, _EMB), jnp.float32),
        compiler_params=pltpu.CompilerParams(
            dimension_semantics=("arbitrary",)),
    )(instance,
      proj_W, proj_b.reshape(1, _EMB), Ws, bs,
      gn_gamma.reshape(1, _EMB), gn_beta.reshape(1, _EMB),
      gn_alpha.reshape(1, _EMB))
    return out


# speedup vs baseline: 1.1018x; 1.1018x over previous
import jax
import jax.numpy as jnp
from jax.experimental import pallas as pl
from jax.experimental.pallas import tpu as pltpu

_EMB = 128
_NLAYERS = 3
_G = 16


def _gnn_body(xc_ref, pw_ref, pb_ref,
              ws_ref, bs_ref, gamma_ref, beta_ref, alpha_ref, out_ref):
    n = xc_ref.shape[1]
    pw = pw_ref[...]
    pb = pb_ref[...]
    gamma = gamma_ref[...]
    beta = beta_ref[...]
    alpha = alpha_ref[...]

    ir = jax.lax.broadcasted_iota(jnp.int32, (n, n), 0)
    ic = jax.lax.broadcasted_iota(jnp.int32, (n, n), 1)
    eye = (ir == ic).astype(jnp.float32)

    adjs, hs = [], []
    for g in range(_G):
        xg = xc_ref[g]
        xt = jnp.transpose(xg)
        x0c = xg[:, 0:1]
        x1c = xg[:, 1:2]
        x0r = xt[0:1, :]
        x1r = xt[1:2, :]
        d0 = x0c - x0r
        d1 = x1c - x1r
        dist = jnp.sqrt(d0 * d0 + d1 * d1)
        wmat = dist + eye
        dinv_c = jax.lax.rsqrt(jnp.sum(wmat, axis=1, keepdims=True))
        dinv_r = jax.lax.rsqrt(jnp.sum(wmat, axis=0, keepdims=True))
        adjs.append(dinv_c * wmat * dinv_r)
        hs.append(x0c * pw[0:1, :] + x1c * pw[1:2, :] + pb)

    inv_n = 1.0 / n
    for l in range(_NLAYERS):
        w_l = ws_ref[l]
        b_l = bs_ref[l:l + 1, :]
        for g in range(_G):
            h = hs[g]
            xw = jnp.dot(h, w_l,
                         preferred_element_type=jnp.float32,
                         precision=jax.lax.Precision.HIGHEST)
            msg = jnp.dot(adjs[g], xw,
                          preferred_element_type=jnp.float32,
                          precision=jax.lax.Precision.HIGHEST)
            h = jnp.tanh(msg + b_l) + h
            mean = jnp.sum(h, axis=0, keepdims=True) * inv_n
            cent = h - alpha * mean
            var = jnp.sum(cent * cent, axis=0, keepdims=True) * inv_n
            hs[g] = gamma * (cent * jax.lax.rsqrt(var + 1e-5)) + beta

    for g in range(_G):
        out_ref[g] = hs[g]


def kernel(instance, proj_W, proj_b, Ws, bs, gn_gamma, gn_beta, gn_alpha):
    b, n, _ = instance.shape
    full = lambda shape: pl.BlockSpec(shape, lambda g: (0,) * len(shape))
    out = pl.pallas_call(
        _gnn_body,
        grid=(b // _G,),
        in_specs=[
            pl.BlockSpec((_G, n, 2), lambda g: (g, 0, 0)),
            full((2, _EMB)),
            full((1, _EMB)),
            full((_NLAYERS, _EMB, _EMB)),
            full((_NLAYERS, _EMB)),
            full((1, _EMB)),
            full((1, _EMB)),
            full((1, _EMB)),
        ],
        out_specs=pl.BlockSpec((_G, n, _EMB), lambda g: (g, 0, 0)),
        out_shape=jax.ShapeDtypeStruct((b, n, _EMB), jnp.float32),
        compiler_params=pltpu.CompilerParams(
            dimension_semantics=("arbitrary",)),
    )(instance,
      proj_W, proj_b.reshape(1, _EMB), Ws, bs,
      gn_gamma.reshape(1, _EMB), gn_beta.reshape(1, _EMB),
      gn_alpha.reshape(1, _EMB))
    return out
